# baseline (speedup 1.0000x reference)
import functools

import jax
import jax.numpy as jnp
from jax import lax
from jax.experimental import pallas as pl
from jax.experimental.pallas import tpu as pltpu
from jax.experimental.pallas import tpu_sc as plsc

N = 10000
E = 320000
D = 128
G = 32

NP = 10240
TN = 1024
NT = NP // TN
K = 128
EP = 327680
EPT = EP // 32
CPT = EPT // K
NBUF = 2
RPT = NP // 16

_mesh = plsc.VectorSubcoreMesh(core_axis_name="c", subcore_axis_name="s")


def _deg_body(ei_hbm, out_hbm, est, onesv, zbuf, deg_sh):
    c = lax.axis_index("c")
    s = lax.axis_index("s")
    for j in range(RPT // 16):
        zbuf[pl.ds(j * 16, 16)] = jnp.zeros((16,), jnp.float32)
    for j in range(K // 16):
        onesv[pl.ds(j * 16, 16)] = jnp.ones((16,), jnp.float32)
    pltpu.sync_copy(zbuf, deg_sh.at[pl.ds(s * RPT, RPT)])
    pltpu.sync_copy(ei_hbm.at[c, s], est)
    plsc.subcore_barrier()

    def body(i, carry):
        pltpu.sync_copy(onesv, deg_sh.at[est.at[i, 1]], add=True)
        return carry

    lax.fori_loop(0, CPT, body, 0)
    plsc.subcore_barrier()
    pltpu.sync_copy(deg_sh.at[pl.ds(s * RPT, RPT)], out_hbm.at[c, pl.ds(s * RPT, RPT)])


_deg_call = pl.kernel(
    _deg_body,
    out_type=jax.ShapeDtypeStruct((2, NP), jnp.float32),
    mesh=_mesh,
    scratch_types=[
        pltpu.VMEM((CPT, 2, K), jnp.int32),
        pltpu.VMEM((K,), jnp.float32),
        pltpu.VMEM((RPT,), jnp.float32),
        pltpu.VMEM_SHARED((NP,), jnp.float32),
    ],
)


def _agg_body(y_hbm, yh_hbm, ei_hbm, p_hbm, idx, rows, acc_sh, sems):
    c = lax.axis_index("c")
    s = lax.axis_index("s")
    r0 = s * RPT
    pltpu.sync_copy(yh_hbm.at[pl.ds(r0, RPT)], acc_sh.at[pl.ds(r0, RPT)])
    plsc.subcore_barrier()

    for b in range(NBUF):
        pltpu.sync_copy(ei_hbm.at[c, s, b], idx.at[b])
        pltpu.async_copy(y_hbm.at[idx.at[b, 0]], rows.at[b], sems.at[b])

    def outer(g, carry):
        i0 = g * NBUF
        for b in range(NBUF):
            i = i0 + b
            pltpu.make_async_copy(y_hbm.at[idx.at[b, 0]], rows.at[b],
                                  sems.at[b]).wait()
            pltpu.sync_copy(rows.at[b], acc_sh.at[idx.at[b, 1]], add=True)

            @pl.when(i + NBUF < CPT)
            def _():
                pltpu.sync_copy(ei_hbm.at[c, s, i + NBUF], idx.at[b])
                pltpu.async_copy(y_hbm.at[idx.at[b, 0]], rows.at[b],
                                 sems.at[b])

        return carry

    lax.fori_loop(0, CPT // NBUF, outer, 0)
    plsc.subcore_barrier()
    pltpu.sync_copy(acc_sh.at[pl.ds(r0, RPT)], p_hbm.at[c, pl.ds(r0, RPT)])


_agg_call = pl.kernel(
    _agg_body,
    out_type=jax.ShapeDtypeStruct((2, NP, D), jnp.float32),
    mesh=_mesh,
    scratch_types=[
        pltpu.VMEM((NBUF, 2, K), jnp.int32),
        pltpu.VMEM((NBUF, K, D), jnp.float32),
        pltpu.VMEM_SHARED((NP, D), jnp.float32),
        pltpu.SemaphoreType.DMA((NBUF,)),
    ],
)


def _gelu(x):
    return 0.5 * x * (1.0 + lax.erf(x * 0.7071067811865476))


def _dinv_body(deg_ref, out_ref):
    out_ref[...] = lax.rsqrt(deg_ref[0:1, :] + deg_ref[1:2, :] + 1.0)


_dinv_call = pl.pallas_call(
    _dinv_body,
    out_shape=jax.ShapeDtypeStruct((1, NP), jnp.float32),
)


def _prep_body(x_ref, w_ref, dinv_ref, y_ref, yh_ref):
    y = dinv_ref[...] * jnp.dot(x_ref[...], w_ref[...],
                                preferred_element_type=jnp.float32)
    y_ref[...] = y
    yh_ref[...] = 0.5 * y


_prep_call = pl.pallas_call(
    _prep_body,
    grid=(NT,),
    in_specs=[
        pl.BlockSpec((TN, D), lambda i: (i, 0)),
        pl.BlockSpec((D, D), lambda i: (0, 0)),
        pl.BlockSpec((TN, 1), lambda i: (i, 0)),
    ],
    out_specs=[
        pl.BlockSpec((TN, D), lambda i: (i, 0)),
        pl.BlockSpec((TN, D), lambda i: (i, 0)),
    ],
    out_shape=[
        jax.ShapeDtypeStruct((NP, D), jnp.float32),
        jax.ShapeDtypeStruct((NP, D), jnp.float32),
    ],
)


def _mid_body(p_ref, dinv_ref, b_ref, w_ref, y_ref, yh_ref):
    dv = dinv_ref[...]
    h = _gelu(dv * (p_ref[0] + p_ref[1]) + b_ref[...])
    y = dv * jnp.dot(h, w_ref[...], preferred_element_type=jnp.float32)
    y_ref[...] = y
    yh_ref[...] = 0.5 * y


_mid_call = pl.pallas_call(
    _mid_body,
    grid=(NT,),
    in_specs=[
        pl.BlockSpec((2, TN, D), lambda i: (0, i, 0)),
        pl.BlockSpec((TN, 1), lambda i: (i, 0)),
        pl.BlockSpec((1, D), lambda i: (0, 0)),
        pl.BlockSpec((D, D), lambda i: (0, 0)),
    ],
    out_specs=[
        pl.BlockSpec((TN, D), lambda i: (i, 0)),
        pl.BlockSpec((TN, D), lambda i: (i, 0)),
    ],
    out_shape=[
        jax.ShapeDtypeStruct((NP, D), jnp.float32),
        jax.ShapeDtypeStruct((NP, D), jnp.float32),
    ],
)


def _final_body(p_ref, dinv_ref, b3_ref, batch_ref, wl1_ref, bl1_ref,
                wl2_ref, bl2_ref, out_ref, acc_s, acc_c):
    i = pl.program_id(0)
    dv = dinv_ref[...]
    h = _gelu(dv * (p_ref[0] + p_ref[1]) + b3_ref[...])
    bt = batch_ref[0]
    gids = lax.broadcasted_iota(jnp.int32, (G, TN), 0)
    oh = (bt == gids).astype(jnp.float32)

    @pl.when(i == 0)
    def _():
        acc_s[...] = jnp.zeros_like(acc_s)
        acc_c[...] = jnp.zeros_like(acc_c)

    acc_s[...] += jnp.dot(oh, h, preferred_element_type=jnp.float32)
    acc_c[...] += jnp.sum(oh, axis=1, keepdims=True)

    @pl.when(i == pl.num_programs(0) - 1)
    def _():
        pooled = acc_s[...] / jnp.maximum(acc_c[...], 1.0)
        t = jnp.dot(pooled, wl1_ref[...],
                    preferred_element_type=jnp.float32) + bl1_ref[...]
        t = jnp.where(t > 0, t, jnp.exp(jnp.minimum(t, 0.0)) - 1.0)
        out_ref[...] = jnp.dot(t, wl2_ref[...],
                               preferred_element_type=jnp.float32) + bl2_ref[...]


_final_call = pl.pallas_call(
    _final_body,
    grid=(NT,),
    in_specs=[
        pl.BlockSpec((2, TN, D), lambda i: (0, i, 0)),
        pl.BlockSpec((TN, 1), lambda i: (i, 0)),
        pl.BlockSpec((1, D), lambda i: (0, 0)),
        pl.BlockSpec((1, 1, TN), lambda i: (i, 0, 0)),
        pl.BlockSpec((D, D // 2), lambda i: (0, 0)),
        pl.BlockSpec((1, D // 2), lambda i: (0, 0)),
        pl.BlockSpec((D // 2, 1), lambda i: (0, 0)),
        pl.BlockSpec((1, 1), lambda i: (0, 0)),
    ],
    out_specs=pl.BlockSpec((G, 1), lambda i: (0, 0)),
    out_shape=jax.ShapeDtypeStruct((G, 1), jnp.float32),
    scratch_shapes=[
        pltpu.VMEM((G, D), jnp.float32),
        pltpu.VMEM((G, 1), jnp.float32),
    ],
)


def kernel(x, edge_index, batch, edge_weight, W1, b1, W2, b2, W3, b3,
           W_lin1, b_lin1, W_lin2, b_lin2):
    src = jnp.pad(edge_index[0].astype(jnp.int32), (0, EP - E),
                  constant_values=0).reshape(2, 16, CPT, K)
    dst = jnp.pad(edge_index[1].astype(jnp.int32), (0, EP - E),
                  constant_values=N).reshape(2, 16, CPT, K)
    ei = jnp.stack([src, dst], axis=3)
    x_p = jnp.pad(x, ((0, NP - N), (0, 0)))
    batch_p = jnp.pad(batch.astype(jnp.int32), (0, NP - N),
                      constant_values=G).reshape(NT, 1, TN)

    deg = _deg_call(ei)
    dinv = _dinv_call(deg).reshape(NP, 1)

    y, yh = _prep_call(x_p, W1, dinv)
    p = _agg_call(y, yh, ei)
    y, yh = _mid_call(p, dinv, b1.reshape(1, D), W2)
    p = _agg_call(y, yh, ei)
    y, yh = _mid_call(p, dinv, b2.reshape(1, D), W3)
    p = _agg_call(y, yh, ei)
    return _final_call(p, dinv, b3.reshape(1, D), batch_p, W_lin1,
                       b_lin1.reshape(1, D // 2), W_lin2,
                       b_lin2.reshape(1, 1))


# speedup vs baseline: 1.2263x; 1.2263x over previous
import functools

import jax
import jax.numpy as jnp
from jax import lax
from jax.experimental import pallas as pl
from jax.experimental.pallas import tpu as pltpu
from jax.experimental.pallas import tpu_sc as plsc

N = 10000
E = 320000
D = 128
G = 32

NP = 10240
TN = 1024
NT = NP // TN
K = 128
EP = 327680
EPT = EP // 32
CPT = EPT // K
NBUF = 2
RPT = NP // 16

_mesh = plsc.VectorSubcoreMesh(core_axis_name="c", subcore_axis_name="s")


def _deg_body(ei_hbm, out_hbm, est, onesv, zbuf, deg_sh):
    c = lax.axis_index("c")
    s = lax.axis_index("s")
    for j in range(RPT // 16):
        zbuf[pl.ds(j * 16, 16)] = jnp.zeros((16,), jnp.float32)
    for j in range(K // 16):
        onesv[pl.ds(j * 16, 16)] = jnp.ones((16,), jnp.float32)
    pltpu.sync_copy(zbuf, deg_sh.at[pl.ds(s * RPT, RPT)])
    pltpu.sync_copy(ei_hbm.at[c, s], est)
    plsc.subcore_barrier()

    def body(i, carry):
        pltpu.sync_copy(onesv, deg_sh.at[est.at[i, 1]], add=True)
        return carry

    lax.fori_loop(0, CPT, body, 0)
    plsc.subcore_barrier()
    pltpu.sync_copy(deg_sh.at[pl.ds(s * RPT, RPT)], out_hbm.at[c, pl.ds(s * RPT, RPT)])


_deg_call = pl.kernel(
    _deg_body,
    out_type=jax.ShapeDtypeStruct((2, NP), jnp.float32),
    mesh=_mesh,
    scratch_types=[
        pltpu.VMEM((CPT, 2, K), jnp.int32),
        pltpu.VMEM((K,), jnp.float32),
        pltpu.VMEM((RPT,), jnp.float32),
        pltpu.VMEM_SHARED((NP,), jnp.float32),
    ],
)


def _agg_body(y_hbm, yh_hbm, ei_hbm, p_hbm, idx, rows, acc_sh, sems):
    c = lax.axis_index("c")
    s = lax.axis_index("s")
    r0 = s * RPT
    pltpu.sync_copy(yh_hbm.at[pl.ds(r0, RPT)], acc_sh.at[pl.ds(r0, RPT)])
    plsc.subcore_barrier()

    for b in range(NBUF):
        pltpu.sync_copy(ei_hbm.at[c, s, b], idx.at[b])
        pltpu.async_copy(y_hbm.at[idx.at[b, 0]], rows.at[b], sems.at[b])

    def outer(g, carry):
        i0 = g * NBUF
        for b in range(NBUF):
            i = i0 + b
            pltpu.make_async_copy(y_hbm.at[idx.at[b, 0]], rows.at[b],
                                  sems.at[b]).wait()
            pltpu.sync_copy(rows.at[b], acc_sh.at[idx.at[b, 1]], add=True)

            @pl.when(i + NBUF < CPT)
            def _():
                pltpu.sync_copy(ei_hbm.at[c, s, i + NBUF], idx.at[b])
                pltpu.async_copy(y_hbm.at[idx.at[b, 0]], rows.at[b],
                                 sems.at[b])

        return carry

    lax.fori_loop(0, CPT // NBUF, outer, 0)
    plsc.subcore_barrier()
    pltpu.sync_copy(acc_sh.at[pl.ds(r0, RPT)], p_hbm.at[c, pl.ds(r0, RPT)])


_agg_call = pl.kernel(
    _agg_body,
    out_type=jax.ShapeDtypeStruct((2, NP, D), jnp.float32),
    mesh=_mesh,
    scratch_types=[
        pltpu.VMEM((NBUF, 2, K), jnp.int32),
        pltpu.VMEM((NBUF, K, D), jnp.float32),
        pltpu.VMEM_SHARED((NP, D), jnp.float32),
        pltpu.SemaphoreType.DMA((NBUF,)),
    ],
)


def _gelu(x):
    return 0.5 * x * (1.0 + lax.erf(x * 0.7071067811865476))


def _dinv_body(deg_ref, out_ref):
    out_ref[...] = lax.rsqrt(deg_ref[0:1, :] + deg_ref[1:2, :] + 1.0)


_dinv_call = pl.pallas_call(
    _dinv_body,
    out_shape=jax.ShapeDtypeStruct((1, NP), jnp.float32),
)


def _prep_body(x_ref, w_ref, dinv_ref, y_ref, yh_ref):
    y = dinv_ref[...] * jnp.dot(x_ref[...], w_ref[...],
                                preferred_element_type=jnp.float32)
    y_ref[...] = y
    yh_ref[...] = 0.5 * y


_prep_call = pl.pallas_call(
    _prep_body,
    grid=(NT,),
    in_specs=[
        pl.BlockSpec((TN, D), lambda i: (i, 0)),
        pl.BlockSpec((D, D), lambda i: (0, 0)),
        pl.BlockSpec((TN, 1), lambda i: (i, 0)),
    ],
    out_specs=[
        pl.BlockSpec((TN, D), lambda i: (i, 0)),
        pl.BlockSpec((TN, D), lambda i: (i, 0)),
    ],
    out_shape=[
        jax.ShapeDtypeStruct((NP, D), jnp.float32),
        jax.ShapeDtypeStruct((NP, D), jnp.float32),
    ],
)


def _mid_body(p_ref, dinv_ref, b_ref, w_ref, y_ref, yh_ref):
    dv = dinv_ref[...]
    h = _gelu(dv * (p_ref[0] + p_ref[1]) + b_ref[...])
    y = dv * jnp.dot(h, w_ref[...], preferred_element_type=jnp.float32)
    y_ref[...] = y
    yh_ref[...] = 0.5 * y


_mid_call = pl.pallas_call(
    _mid_body,
    grid=(NT,),
    in_specs=[
        pl.BlockSpec((2, TN, D), lambda i: (0, i, 0)),
        pl.BlockSpec((TN, 1), lambda i: (i, 0)),
        pl.BlockSpec((1, D), lambda i: (0, 0)),
        pl.BlockSpec((D, D), lambda i: (0, 0)),
    ],
    out_specs=[
        pl.BlockSpec((TN, D), lambda i: (i, 0)),
        pl.BlockSpec((TN, D), lambda i: (i, 0)),
    ],
    out_shape=[
        jax.ShapeDtypeStruct((NP, D), jnp.float32),
        jax.ShapeDtypeStruct((NP, D), jnp.float32),
    ],
)


def _final_body(p_ref, dinv_ref, b3_ref, batch_ref, wl1_ref, bl1_ref,
                wl2_ref, bl2_ref, out_ref, acc_s, acc_c):
    i = pl.program_id(0)
    dv = dinv_ref[...]
    h = _gelu(dv * (p_ref[0] + p_ref[1]) + b3_ref[...])
    bt = batch_ref[0]
    gids = lax.broadcasted_iota(jnp.int32, (G, TN), 0)
    oh = (bt == gids).astype(jnp.float32)

    @pl.when(i == 0)
    def _():
        acc_s[...] = jnp.zeros_like(acc_s)
        acc_c[...] = jnp.zeros_like(acc_c)

    acc_s[...] += jnp.dot(oh, h, preferred_element_type=jnp.float32)
    acc_c[...] += jnp.sum(oh, axis=1, keepdims=True)

    @pl.when(i == pl.num_programs(0) - 1)
    def _():
        pooled = acc_s[...] / jnp.maximum(acc_c[...], 1.0)
        t = jnp.dot(pooled, wl1_ref[...],
                    preferred_element_type=jnp.float32) + bl1_ref[...]
        t = jnp.where(t > 0, t, jnp.exp(jnp.minimum(t, 0.0)) - 1.0)
        out_ref[...] = jnp.dot(t, wl2_ref[...],
                               preferred_element_type=jnp.float32) + bl2_ref[...]


_final_call = pl.pallas_call(
    _final_body,
    grid=(NT,),
    in_specs=[
        pl.BlockSpec((2, TN, D), lambda i: (0, i, 0)),
        pl.BlockSpec((TN, 1), lambda i: (i, 0)),
        pl.BlockSpec((1, D), lambda i: (0, 0)),
        pl.BlockSpec((1, 1, TN), lambda i: (i, 0, 0)),
        pl.BlockSpec((D, D // 2), lambda i: (0, 0)),
        pl.BlockSpec((1, D // 2), lambda i: (0, 0)),
        pl.BlockSpec((D // 2, 1), lambda i: (0, 0)),
        pl.BlockSpec((1, 1), lambda i: (0, 0)),
    ],
    out_specs=pl.BlockSpec((G, 1), lambda i: (0, 0)),
    out_shape=jax.ShapeDtypeStruct((G, 1), jnp.float32),
    scratch_shapes=[
        pltpu.VMEM((G, D), jnp.float32),
        pltpu.VMEM((G, 1), jnp.float32),
    ],
)


def kernel(x, edge_index, batch, edge_weight, W1, b1, W2, b2, W3, b3,
           W_lin1, b_lin1, W_lin2, b_lin2):
    ppt = EPT - E // 32
    src = jnp.concatenate(
        [edge_index[0].astype(jnp.int32).reshape(2, 16, E // 32),
         jnp.zeros((2, 16, ppt), jnp.int32)], axis=2).reshape(2, 16, CPT, K)
    pad_dst = jnp.broadcast_to(N + jnp.arange(ppt, dtype=jnp.int32),
                               (2, 16, ppt))
    dst = jnp.concatenate(
        [edge_index[1].astype(jnp.int32).reshape(2, 16, E // 32),
         pad_dst], axis=2).reshape(2, 16, CPT, K)
    ei = jnp.stack([src, dst], axis=3)
    x_p = jnp.pad(x, ((0, NP - N), (0, 0)))
    batch_p = jnp.pad(batch.astype(jnp.int32), (0, NP - N),
                      constant_values=G).reshape(NT, 1, TN)

    deg = _deg_call(ei)
    dinv = _dinv_call(deg).reshape(NP, 1)

    y, yh = _prep_call(x_p, W1, dinv)
    p = _agg_call(y, yh, ei)
    y, yh = _mid_call(p, dinv, b1.reshape(1, D), W2)
    p = _agg_call(y, yh, ei)
    y, yh = _mid_call(p, dinv, b2.reshape(1, D), W3)
    p = _agg_call(y, yh, ei)
    return _final_call(p, dinv, b3.reshape(1, D), batch_p, W_lin1,
                       b_lin1.reshape(1, D // 2), W_lin2,
                       b_lin2.reshape(1, 1))


# speedup vs baseline: 1.2264x; 1.0001x over previous
import functools

import jax
import jax.numpy as jnp
from jax import lax
from jax.experimental import pallas as pl
from jax.experimental.pallas import tpu as pltpu
from jax.experimental.pallas import tpu_sc as plsc

N = 10000
E = 320000
D = 128
G = 32

NP = 10240
TN = 1024
NT = NP // TN
K = 128
EP = 327680
EPT = EP // 32
CPT = EPT // K
NBUF = 2
RPT = NP // 16

_mesh = plsc.VectorSubcoreMesh(core_axis_name="c", subcore_axis_name="s")


def _deg_body(ei_hbm, out_hbm, est, onesv, zbuf, deg_sh):
    c = lax.axis_index("c")
    s = lax.axis_index("s")
    for j in range(RPT // 16):
        zbuf[pl.ds(j * 16, 16)] = jnp.zeros((16,), jnp.float32)
    for j in range(K // 16):
        onesv[pl.ds(j * 16, 16)] = jnp.ones((16,), jnp.float32)
    pltpu.sync_copy(zbuf, deg_sh.at[pl.ds(s * RPT, RPT)])
    pltpu.sync_copy(ei_hbm.at[c, s], est)
    plsc.subcore_barrier()

    def body(i, carry):
        pltpu.sync_copy(onesv, deg_sh.at[est.at[i, 1]], add=True)
        return carry

    lax.fori_loop(0, CPT, body, 0)
    plsc.subcore_barrier()
    pltpu.sync_copy(deg_sh.at[pl.ds(s * RPT, RPT)], out_hbm.at[c, pl.ds(s * RPT, RPT)])


_deg_call = pl.kernel(
    _deg_body,
    out_type=jax.ShapeDtypeStruct((2, NP), jnp.float32),
    mesh=_mesh,
    scratch_types=[
        pltpu.VMEM((CPT, 2, K), jnp.int32),
        pltpu.VMEM((K,), jnp.float32),
        pltpu.VMEM((RPT,), jnp.float32),
        pltpu.VMEM_SHARED((NP,), jnp.float32),
    ],
)


def _agg_body(y_hbm, yh_hbm, ei_hbm, p_hbm, idx, rows, acc_sh, sems):
    c = lax.axis_index("c")
    s = lax.axis_index("s")
    r0 = s * RPT
    pltpu.sync_copy(yh_hbm.at[pl.ds(r0, RPT)], acc_sh.at[pl.ds(r0, RPT)])
    plsc.subcore_barrier()

    for b in range(NBUF):
        pltpu.sync_copy(ei_hbm.at[c, s, b], idx.at[b])
        pltpu.async_copy(y_hbm.at[idx.at[b, 0]], rows.at[b], sems.at[b])

    def outer(g, carry):
        i0 = g * NBUF
        for b in range(NBUF):
            i = i0 + b
            pltpu.make_async_copy(y_hbm.at[idx.at[b, 0]], rows.at[b],
                                  sems.at[b]).wait()
            pltpu.sync_copy(rows.at[b], acc_sh.at[idx.at[b, 1]], add=True)

            @pl.when(i + NBUF < CPT)
            def _():
                pltpu.sync_copy(ei_hbm.at[c, s, i + NBUF], idx.at[b])
                pltpu.async_copy(y_hbm.at[idx.at[b, 0]], rows.at[b],
                                 sems.at[b])

        return carry

    lax.fori_loop(0, CPT // NBUF, outer, 0)
    plsc.subcore_barrier()
    pltpu.sync_copy(acc_sh.at[pl.ds(r0, RPT)], p_hbm.at[c, pl.ds(r0, RPT)])


_agg_call = pl.kernel(
    _agg_body,
    out_type=jax.ShapeDtypeStruct((2, NP, D), jnp.float32),
    mesh=_mesh,
    scratch_types=[
        pltpu.VMEM((NBUF, 2, K), jnp.int32),
        pltpu.VMEM((NBUF, K, D), jnp.float32),
        pltpu.VMEM_SHARED((NP, D), jnp.float32),
        pltpu.SemaphoreType.DMA((NBUF,)),
    ],
)


def _gelu(x):
    return 0.5 * x * (1.0 + lax.erf(x * 0.7071067811865476))


def _dinv_body(deg_ref, out_ref):
    out_ref[...] = lax.rsqrt(deg_ref[0:1, :] + deg_ref[1:2, :] + 1.0)


_dinv_call = pl.pallas_call(
    _dinv_body,
    out_shape=jax.ShapeDtypeStruct((1, NP), jnp.float32),
)


def _prep_body(x_ref, w_ref, dinv_ref, y_ref, yh_ref):
    y = dinv_ref[...] * jnp.dot(x_ref[...], w_ref[...],
                                preferred_element_type=jnp.float32)
    y_ref[...] = y
    yh_ref[...] = 0.5 * y


_prep_call = pl.pallas_call(
    _prep_body,
    grid=(NT,),
    in_specs=[
        pl.BlockSpec((TN, D), lambda i: (i, 0)),
        pl.BlockSpec((D, D), lambda i: (0, 0)),
        pl.BlockSpec((TN, 1), lambda i: (i, 0)),
    ],
    out_specs=[
        pl.BlockSpec((TN, D), lambda i: (i, 0)),
        pl.BlockSpec((TN, D), lambda i: (i, 0)),
    ],
    out_shape=[
        jax.ShapeDtypeStruct((NP, D), jnp.float32),
        jax.ShapeDtypeStruct((NP, D), jnp.float32),
    ],
)


def _mid_body(p_ref, dinv_ref, b_ref, w_ref, y_ref, yh_ref):
    dv = dinv_ref[...]
    h = _gelu(dv * (p_ref[0] + p_ref[1]) + b_ref[...])
    y = dv * jnp.dot(h, w_ref[...], preferred_element_type=jnp.float32)
    y_ref[...] = y
    yh_ref[...] = 0.5 * y


_mid_call = pl.pallas_call(
    _mid_body,
    grid=(NT,),
    in_specs=[
        pl.BlockSpec((2, TN, D), lambda i: (0, i, 0)),
        pl.BlockSpec((TN, 1), lambda i: (i, 0)),
        pl.BlockSpec((1, D), lambda i: (0, 0)),
        pl.BlockSpec((D, D), lambda i: (0, 0)),
    ],
    out_specs=[
        pl.BlockSpec((TN, D), lambda i: (i, 0)),
        pl.BlockSpec((TN, D), lambda i: (i, 0)),
    ],
    out_shape=[
        jax.ShapeDtypeStruct((NP, D), jnp.float32),
        jax.ShapeDtypeStruct((NP, D), jnp.float32),
    ],
)


def _final_body(p_ref, dinv_ref, b3_ref, batch_ref, wl1_ref, bl1_ref,
                wl2_ref, bl2_ref, out_ref, acc_s, acc_c):
    i = pl.program_id(0)
    dv = dinv_ref[...]
    h = _gelu(dv * (p_ref[0] + p_ref[1]) + b3_ref[...])
    bt = batch_ref[0]
    gids = lax.broadcasted_iota(jnp.int32, (G, TN), 0)
    oh = (bt == gids).astype(jnp.float32)

    @pl.when(i == 0)
    def _():
        acc_s[...] = jnp.zeros_like(acc_s)
        acc_c[...] = jnp.zeros_like(acc_c)

    acc_s[...] += jnp.dot(oh, h, preferred_element_type=jnp.float32)
    acc_c[...] += jnp.sum(oh, axis=1, keepdims=True)

    @pl.when(i == pl.num_programs(0) - 1)
    def _():
        pooled = acc_s[...] / jnp.maximum(acc_c[...], 1.0)
        t = jnp.dot(pooled, wl1_ref[...],
                    preferred_element_type=jnp.float32) + bl1_ref[...]
        t = jnp.where(t > 0, t, jnp.exp(jnp.minimum(t, 0.0)) - 1.0)
        out_ref[...] = jnp.dot(t, wl2_ref[...],
                               preferred_element_type=jnp.float32) + bl2_ref[...]


_final_call = pl.pallas_call(
    _final_body,
    grid=(NT,),
    in_specs=[
        pl.BlockSpec((2, TN, D), lambda i: (0, i, 0)),
        pl.BlockSpec((TN, 1), lambda i: (i, 0)),
        pl.BlockSpec((1, D), lambda i: (0, 0)),
        pl.BlockSpec((1, 1, TN), lambda i: (i, 0, 0)),
        pl.BlockSpec((D, D // 2), lambda i: (0, 0)),
        pl.BlockSpec((1, D // 2), lambda i: (0, 0)),
        pl.BlockSpec((D // 2, 1), lambda i: (0, 0)),
        pl.BlockSpec((1, 1), lambda i: (0, 0)),
    ],
    out_specs=pl.BlockSpec((G, 1), lambda i: (0, 0)),
    out_shape=jax.ShapeDtypeStruct((G, 1), jnp.float32),
    scratch_shapes=[
        pltpu.VMEM((G, D), jnp.float32),
        pltpu.VMEM((G, 1), jnp.float32),
    ],
)


def kernel(x, edge_index, batch, edge_weight, W1, b1, W2, b2, W3, b3,
           W_lin1, b_lin1, W_lin2, b_lin2):
    ppt = EPT - E // 32
    src = jnp.concatenate(
        [edge_index[0].astype(jnp.int32).reshape(2, 16, E // 32),
         jnp.zeros((2, 16, ppt), jnp.int32)], axis=2).reshape(2, 16, CPT, K)
    pad_dst = jnp.broadcast_to(
        N + (jnp.arange(ppt, dtype=jnp.int32)[None, :]
             + (ppt // 16) * jnp.arange(16, dtype=jnp.int32)[:, None]) % ppt,
        (2, 16, ppt))
    dst = jnp.concatenate(
        [edge_index[1].astype(jnp.int32).reshape(2, 16, E // 32),
         pad_dst], axis=2).reshape(2, 16, CPT, K)
    ei = jnp.stack([src, dst], axis=3)
    x_p = jnp.pad(x, ((0, NP - N), (0, 0)))
    batch_p = jnp.pad(batch.astype(jnp.int32), (0, NP - N),
                      constant_values=G).reshape(NT, 1, TN)

    deg = _deg_call(ei)
    dinv = _dinv_call(deg).reshape(NP, 1)

    y, yh = _prep_call(x_p, W1, dinv)
    p = _agg_call(y, yh, ei)
    y, yh = _mid_call(p, dinv, b1.reshape(1, D), W2)
    p = _agg_call(y, yh, ei)
    y, yh = _mid_call(p, dinv, b2.reshape(1, D), W3)
    p = _agg_call(y, yh, ei)
    return _final_call(p, dinv, b3.reshape(1, D), batch_p, W_lin1,
                       b_lin1.reshape(1, D // 2), W_lin2,
                       b_lin2.reshape(1, 1))


# speedup vs baseline: 3.4132x; 2.7832x over previous
import functools

import jax
import jax.numpy as jnp
from jax import lax
from jax.experimental import pallas as pl
from jax.experimental.pallas import tpu as pltpu
from jax.experimental.pallas import tpu_sc as plsc

N = 10000
E = 320000
D = 128
G = 32

NP = 10240
TN = 1024
NT = NP // TN
K = 128
EP = 327680
EPT = EP // 32
CPT = EPT // K
NBUF = 2
RPT = NP // 16

_mesh = plsc.VectorSubcoreMesh(core_axis_name="c", subcore_axis_name="s")


def _deg_body(ei_hbm, out_hbm, est, onesv, zbuf, deg_sh):
    c = lax.axis_index("c")
    s = lax.axis_index("s")
    for j in range(RPT // 16):
        zbuf[pl.ds(j * 16, 16)] = jnp.zeros((16,), jnp.float32)
    for j in range(K // 16):
        onesv[pl.ds(j * 16, 16)] = jnp.ones((16,), jnp.float32)
    pltpu.sync_copy(zbuf, deg_sh.at[pl.ds(s * RPT, RPT)])
    pltpu.sync_copy(ei_hbm.at[c, s], est)
    plsc.subcore_barrier()

    def body(i, carry):
        pltpu.sync_copy(onesv, deg_sh.at[est.at[i, 1]], add=True)
        return carry

    lax.fori_loop(0, CPT, body, 0)
    plsc.subcore_barrier()
    pltpu.sync_copy(deg_sh.at[pl.ds(s * RPT, RPT)], out_hbm.at[c, pl.ds(s * RPT, RPT)])


_deg_call = pl.kernel(
    _deg_body,
    out_type=jax.ShapeDtypeStruct((2, NP), jnp.float32),
    mesh=_mesh,
    scratch_types=[
        pltpu.VMEM((CPT, 2, K), jnp.int32),
        pltpu.VMEM((K,), jnp.float32),
        pltpu.VMEM((RPT,), jnp.float32),
        pltpu.VMEM_SHARED((NP,), jnp.float32),
    ],
)


def _agg_body(y_hbm, yh_hbm, ei_hbm, p_hbm, idx, rows, acc_sh, sems):
    c = lax.axis_index("c")
    s = lax.axis_index("s")
    r0 = s * RPT
    pltpu.sync_copy(yh_hbm.at[pl.ds(r0, RPT)], acc_sh.at[pl.ds(r0, RPT)])
    plsc.subcore_barrier()

    for b in range(NBUF):
        pltpu.sync_copy(ei_hbm.at[c, s, b], idx.at[b])
        pltpu.async_copy(y_hbm.at[idx.at[b, 0]], rows.at[b], sems.at[b])

    def outer(g, carry):
        i0 = g * NBUF
        for b in range(NBUF):
            i = i0 + b
            pltpu.make_async_copy(y_hbm.at[idx.at[b, 0]], rows.at[b],
                                  sems.at[b]).wait()
            pltpu.sync_copy(rows.at[b], acc_sh.at[idx.at[b, 1]], add=True)

            @pl.when(i + NBUF < CPT)
            def _():
                pltpu.sync_copy(ei_hbm.at[c, s, i + NBUF], idx.at[b])
                pltpu.async_copy(y_hbm.at[idx.at[b, 0]], rows.at[b],
                                 sems.at[b])

        return carry

    lax.fori_loop(0, CPT // NBUF, outer, 0)
    plsc.subcore_barrier()
    pltpu.sync_copy(acc_sh.at[pl.ds(r0, RPT)], p_hbm.at[c, pl.ds(r0, RPT)])


_agg_call = pl.kernel(
    _agg_body,
    out_type=jax.ShapeDtypeStruct((2, NP, D), jnp.float32),
    mesh=_mesh,
    scratch_types=[
        pltpu.VMEM((NBUF, 2, K), jnp.int32),
        pltpu.VMEM((NBUF, K, D), jnp.float32),
        pltpu.VMEM_SHARED((NP, D), jnp.float32),
        pltpu.SemaphoreType.DMA((NBUF,)),
    ],
)


def _gelu(x):
    return 0.5 * x * (1.0 + lax.erf(x * 0.7071067811865476))


def _dinv_body(deg_ref, out_ref):
    out_ref[...] = lax.rsqrt(deg_ref[0:1, :] + deg_ref[1:2, :] + 1.0)


_dinv_call = pl.pallas_call(
    _dinv_body,
    out_shape=jax.ShapeDtypeStruct((1, NP), jnp.float32),
)


def _prep_body(x_ref, w_ref, dinv_ref, y_ref, yh_ref):
    y = dinv_ref[...] * jnp.dot(x_ref[...], w_ref[...],
                                preferred_element_type=jnp.float32)
    y_ref[...] = y
    yh_ref[...] = 0.5 * y


_prep_call = pl.pallas_call(
    _prep_body,
    grid=(NT,),
    in_specs=[
        pl.BlockSpec((TN, D), lambda i: (i, 0)),
        pl.BlockSpec((D, D), lambda i: (0, 0)),
        pl.BlockSpec((TN, 1), lambda i: (i, 0)),
    ],
    out_specs=[
        pl.BlockSpec((TN, D), lambda i: (i, 0)),
        pl.BlockSpec((TN, D), lambda i: (i, 0)),
    ],
    out_shape=[
        jax.ShapeDtypeStruct((NP, D), jnp.float32),
        jax.ShapeDtypeStruct((NP, D), jnp.float32),
    ],
)


def _mid_body(p_ref, dinv_ref, b_ref, w_ref, y_ref, yh_ref):
    dv = dinv_ref[...]
    h = _gelu(dv * (p_ref[0] + p_ref[1]) + b_ref[...])
    y = dv * jnp.dot(h, w_ref[...], preferred_element_type=jnp.float32)
    y_ref[...] = y
    yh_ref[...] = 0.5 * y


_mid_call = pl.pallas_call(
    _mid_body,
    grid=(NT,),
    in_specs=[
        pl.BlockSpec((2, TN, D), lambda i: (0, i, 0)),
        pl.BlockSpec((TN, 1), lambda i: (i, 0)),
        pl.BlockSpec((1, D), lambda i: (0, 0)),
        pl.BlockSpec((D, D), lambda i: (0, 0)),
    ],
    out_specs=[
        pl.BlockSpec((TN, D), lambda i: (i, 0)),
        pl.BlockSpec((TN, D), lambda i: (i, 0)),
    ],
    out_shape=[
        jax.ShapeDtypeStruct((NP, D), jnp.float32),
        jax.ShapeDtypeStruct((NP, D), jnp.float32),
    ],
)


def _final_body(p_ref, dinv_ref, b3_ref, batch_ref, wl1_ref, bl1_ref,
                wl2_ref, bl2_ref, out_ref, acc_s, acc_c):
    i = pl.program_id(0)
    dv = dinv_ref[...]
    h = _gelu(dv * (p_ref[0] + p_ref[1]) + b3_ref[...])
    bt = batch_ref[0]
    gids = lax.broadcasted_iota(jnp.int32, (G, TN), 0)
    oh = (bt == gids).astype(jnp.float32)

    @pl.when(i == 0)
    def _():
        acc_s[...] = jnp.zeros_like(acc_s)
        acc_c[...] = jnp.zeros_like(acc_c)

    acc_s[...] += jnp.dot(oh, h, preferred_element_type=jnp.float32)
    acc_c[...] += jnp.sum(oh, axis=1, keepdims=True)

    @pl.when(i == pl.num_programs(0) - 1)
    def _():
        pooled = acc_s[...] / jnp.maximum(acc_c[...], 1.0)
        t = jnp.dot(pooled, wl1_ref[...],
                    preferred_element_type=jnp.float32) + bl1_ref[...]
        t = jnp.where(t > 0, t, jnp.exp(jnp.minimum(t, 0.0)) - 1.0)
        out_ref[...] = jnp.dot(t, wl2_ref[...],
                               preferred_element_type=jnp.float32) + bl2_ref[...]


_final_call = pl.pallas_call(
    _final_body,
    grid=(NT,),
    in_specs=[
        pl.BlockSpec((2, TN, D), lambda i: (0, i, 0)),
        pl.BlockSpec((TN, 1), lambda i: (i, 0)),
        pl.BlockSpec((1, D), lambda i: (0, 0)),
        pl.BlockSpec((1, 1, TN), lambda i: (i, 0, 0)),
        pl.BlockSpec((D, D // 2), lambda i: (0, 0)),
        pl.BlockSpec((1, D // 2), lambda i: (0, 0)),
        pl.BlockSpec((D // 2, 1), lambda i: (0, 0)),
        pl.BlockSpec((1, 1), lambda i: (0, 0)),
    ],
    out_specs=pl.BlockSpec((G, 1), lambda i: (0, 0)),
    out_shape=jax.ShapeDtypeStruct((G, 1), jnp.float32),
    scratch_shapes=[
        pltpu.VMEM((G, D), jnp.float32),
        pltpu.VMEM((G, 1), jnp.float32),
    ],
)


def kernel(x, edge_index, batch, edge_weight, W1, b1, W2, b2, W3, b3,
           W_lin1, b_lin1, W_lin2, b_lin2):
    ppt = EPT - E // 32
    pad_src = jnp.broadcast_to(
        (jnp.arange(ppt, dtype=jnp.int32)[None, :] * 37
         + 613 * jnp.arange(16, dtype=jnp.int32)[:, None]) % N,
        (2, 16, ppt))
    src = jnp.concatenate(
        [edge_index[0].astype(jnp.int32).reshape(2, 16, E // 32),
         pad_src], axis=2).reshape(2, 16, CPT, K)
    pad_dst = jnp.broadcast_to(
        N + (jnp.arange(ppt, dtype=jnp.int32)[None, :]
             + (ppt // 16) * jnp.arange(16, dtype=jnp.int32)[:, None]) % ppt,
        (2, 16, ppt))
    dst = jnp.concatenate(
        [edge_index[1].astype(jnp.int32).reshape(2, 16, E // 32),
         pad_dst], axis=2).reshape(2, 16, CPT, K)
    ei = jnp.stack([src, dst], axis=3)
    x_p = jnp.pad(x, ((0, NP - N), (0, 0)))
    batch_p = jnp.pad(batch.astype(jnp.int32), (0, NP - N),
                      constant_values=G).reshape(NT, 1, TN)

    deg = _deg_call(ei)
    dinv = _dinv_call(deg).reshape(NP, 1)

    y, yh = _prep_call(x_p, W1, dinv)
    p = _agg_call(y, yh, ei)
    y, yh = _mid_call(p, dinv, b1.reshape(1, D), W2)
    p = _agg_call(y, yh, ei)
    y, yh = _mid_call(p, dinv, b2.reshape(1, D), W3)
    p = _agg_call(y, yh, ei)
    return _final_call(p, dinv, b3.reshape(1, D), batch_p, W_lin1,
                       b_lin1.reshape(1, D // 2), W_lin2,
                       b_lin2.reshape(1, 1))


# speedup vs baseline: 3.8511x; 1.1283x over previous
import functools

import jax
import jax.numpy as jnp
from jax import lax
from jax.experimental import pallas as pl
from jax.experimental.pallas import tpu as pltpu
from jax.experimental.pallas import tpu_sc as plsc

N = 10000
E = 320000
D = 128
G = 32

NP = 10240
TN = 1024
NT = NP // TN
K = 128
EP = 327680
EPT = EP // 32
CPT = EPT // K
NBUF = 2
RPT = NP // 16

_mesh = plsc.VectorSubcoreMesh(core_axis_name="c", subcore_axis_name="s")


def _deg_body(ei_hbm, out_hbm, est, onesv, zbuf, deg_sh):
    c = lax.axis_index("c")
    s = lax.axis_index("s")
    for j in range(RPT // 16):
        zbuf[pl.ds(j * 16, 16)] = jnp.zeros((16,), jnp.float32)
    for j in range(K // 16):
        onesv[pl.ds(j * 16, 16)] = jnp.ones((16,), jnp.float32)
    pltpu.sync_copy(zbuf, deg_sh.at[pl.ds(s * RPT, RPT)])
    pltpu.sync_copy(ei_hbm.at[c, s], est)
    plsc.subcore_barrier()

    def body(i, carry):
        pltpu.sync_copy(onesv, deg_sh.at[est.at[i, 0]], add=True)
        return carry

    lax.fori_loop(0, CPT, body, 0)
    plsc.subcore_barrier()
    pltpu.sync_copy(deg_sh.at[pl.ds(s * RPT, RPT)], out_hbm.at[c, pl.ds(s * RPT, RPT)])


_deg_call = pl.kernel(
    _deg_body,
    out_type=jax.ShapeDtypeStruct((2, NP), jnp.float32),
    mesh=_mesh,
    scratch_types=[
        pltpu.VMEM((CPT, 1, K), jnp.int32),
        pltpu.VMEM((K,), jnp.float32),
        pltpu.VMEM((RPT,), jnp.float32),
        pltpu.VMEM_SHARED((NP,), jnp.float32),
    ],
)


def _agg_body(y_hbm, yh_hbm, src_hbm, dst_hbm, p_hbm, src_st, dstv, rows,
              acc_sh, gsem, dsem):
    c = lax.axis_index("c")
    s = lax.axis_index("s")
    r0 = s * RPT
    pltpu.sync_copy(src_hbm.at[c, s], src_st)
    pltpu.sync_copy(yh_hbm.at[pl.ds(r0, RPT)], acc_sh.at[pl.ds(r0, RPT)])
    plsc.subcore_barrier()

    for b in range(NBUF):
        pltpu.async_copy(dst_hbm.at[c, s, b], dstv.at[b], dsem.at[b])
        pltpu.async_copy(y_hbm.at[src_st.at[b, 0]], rows.at[b], gsem.at[b])

    def outer(g, carry):
        i0 = g * NBUF
        for b in range(NBUF):
            i = i0 + b
            pltpu.make_async_copy(y_hbm.at[src_st.at[i, 0]], rows.at[b],
                                  gsem.at[b]).wait()
            pltpu.make_async_copy(dst_hbm.at[c, s, i], dstv.at[b],
                                  dsem.at[b]).wait()
            pltpu.sync_copy(rows.at[b], acc_sh.at[dstv.at[b, 0]], add=True)

            @pl.when(i + NBUF < CPT)
            def _():
                pltpu.async_copy(dst_hbm.at[c, s, i + NBUF], dstv.at[b],
                                 dsem.at[b])
                pltpu.async_copy(y_hbm.at[src_st.at[i + NBUF, 0]], rows.at[b],
                                 gsem.at[b])

        return carry

    lax.fori_loop(0, CPT // NBUF, outer, 0)
    plsc.subcore_barrier()
    pltpu.sync_copy(acc_sh.at[pl.ds(r0, RPT)], p_hbm.at[c, pl.ds(r0, RPT)])


_agg_call = pl.kernel(
    _agg_body,
    out_type=jax.ShapeDtypeStruct((2, NP, D), jnp.float32),
    mesh=_mesh,
    scratch_types=[
        pltpu.VMEM((CPT, 1, K), jnp.int32),
        pltpu.VMEM((NBUF, 1, K), jnp.int32),
        pltpu.VMEM((NBUF, K, D), jnp.float32),
        pltpu.VMEM_SHARED((NP, D), jnp.float32),
        pltpu.SemaphoreType.DMA((NBUF,)),
        pltpu.SemaphoreType.DMA((NBUF,)),
    ],
)


def _gelu(x):
    return 0.5 * x * (1.0 + lax.erf(x * 0.7071067811865476))


def _dinv_body(deg_ref, out_ref):
    out_ref[...] = lax.rsqrt(deg_ref[0:1, :] + deg_ref[1:2, :] + 1.0)


_dinv_call = pl.pallas_call(
    _dinv_body,
    out_shape=jax.ShapeDtypeStruct((1, NP), jnp.float32),
)


def _prep_body(x_ref, w_ref, dinv_ref, y_ref, yh_ref):
    y = dinv_ref[...] * jnp.dot(x_ref[...], w_ref[...],
                                preferred_element_type=jnp.float32)
    y_ref[...] = y
    yh_ref[...] = 0.5 * y


_prep_call = pl.pallas_call(
    _prep_body,
    grid=(NT,),
    in_specs=[
        pl.BlockSpec((TN, D), lambda i: (i, 0)),
        pl.BlockSpec((D, D), lambda i: (0, 0)),
        pl.BlockSpec((TN, 1), lambda i: (i, 0)),
    ],
    out_specs=[
        pl.BlockSpec((TN, D), lambda i: (i, 0)),
        pl.BlockSpec((TN, D), lambda i: (i, 0)),
    ],
    out_shape=[
        jax.ShapeDtypeStruct((NP, D), jnp.float32),
        jax.ShapeDtypeStruct((NP, D), jnp.float32),
    ],
)


def _mid_body(p_ref, dinv_ref, b_ref, w_ref, y_ref, yh_ref):
    dv = dinv_ref[...]
    h = _gelu(dv * (p_ref[0] + p_ref[1]) + b_ref[...])
    y = dv * jnp.dot(h, w_ref[...], preferred_element_type=jnp.float32)
    y_ref[...] = y
    yh_ref[...] = 0.5 * y


_mid_call = pl.pallas_call(
    _mid_body,
    grid=(NT,),
    in_specs=[
        pl.BlockSpec((2, TN, D), lambda i: (0, i, 0)),
        pl.BlockSpec((TN, 1), lambda i: (i, 0)),
        pl.BlockSpec((1, D), lambda i: (0, 0)),
        pl.BlockSpec((D, D), lambda i: (0, 0)),
    ],
    out_specs=[
        pl.BlockSpec((TN, D), lambda i: (i, 0)),
        pl.BlockSpec((TN, D), lambda i: (i, 0)),
    ],
    out_shape=[
        jax.ShapeDtypeStruct((NP, D), jnp.float32),
        jax.ShapeDtypeStruct((NP, D), jnp.float32),
    ],
)


def _final_body(p_ref, dinv_ref, b3_ref, batch_ref, wl1_ref, bl1_ref,
                wl2_ref, bl2_ref, out_ref, acc_s, acc_c):
    i = pl.program_id(0)
    dv = dinv_ref[...]
    h = _gelu(dv * (p_ref[0] + p_ref[1]) + b3_ref[...])
    bt = batch_ref[0]
    gids = lax.broadcasted_iota(jnp.int32, (G, TN), 0)
    oh = (bt == gids).astype(jnp.float32)

    @pl.when(i == 0)
    def _():
        acc_s[...] = jnp.zeros_like(acc_s)
        acc_c[...] = jnp.zeros_like(acc_c)

    acc_s[...] += jnp.dot(oh, h, preferred_element_type=jnp.float32)
    acc_c[...] += jnp.sum(oh, axis=1, keepdims=True)

    @pl.when(i == pl.num_programs(0) - 1)
    def _():
        pooled = acc_s[...] / jnp.maximum(acc_c[...], 1.0)
        t = jnp.dot(pooled, wl1_ref[...],
                    preferred_element_type=jnp.float32) + bl1_ref[...]
        t = jnp.where(t > 0, t, jnp.exp(jnp.minimum(t, 0.0)) - 1.0)
        out_ref[...] = jnp.dot(t, wl2_ref[...],
                               preferred_element_type=jnp.float32) + bl2_ref[...]


_final_call = pl.pallas_call(
    _final_body,
    grid=(NT,),
    in_specs=[
        pl.BlockSpec((2, TN, D), lambda i: (0, i, 0)),
        pl.BlockSpec((TN, 1), lambda i: (i, 0)),
        pl.BlockSpec((1, D), lambda i: (0, 0)),
        pl.BlockSpec((1, 1, TN), lambda i: (i, 0, 0)),
        pl.BlockSpec((D, D // 2), lambda i: (0, 0)),
        pl.BlockSpec((1, D // 2), lambda i: (0, 0)),
        pl.BlockSpec((D // 2, 1), lambda i: (0, 0)),
        pl.BlockSpec((1, 1), lambda i: (0, 0)),
    ],
    out_specs=pl.BlockSpec((G, 1), lambda i: (0, 0)),
    out_shape=jax.ShapeDtypeStruct((G, 1), jnp.float32),
    scratch_shapes=[
        pltpu.VMEM((G, D), jnp.float32),
        pltpu.VMEM((G, 1), jnp.float32),
    ],
)


def kernel(x, edge_index, batch, edge_weight, W1, b1, W2, b2, W3, b3,
           W_lin1, b_lin1, W_lin2, b_lin2):
    ppt = EPT - E // 32
    pad_src = jnp.broadcast_to(
        (jnp.arange(ppt, dtype=jnp.int32)[None, :] * 37
         + 613 * jnp.arange(16, dtype=jnp.int32)[:, None]) % N,
        (2, 16, ppt))
    src = jnp.concatenate(
        [edge_index[0].astype(jnp.int32).reshape(2, 16, E // 32),
         pad_src], axis=2).reshape(2, 16, CPT, 1, K)
    pad_dst = jnp.broadcast_to(
        N + (jnp.arange(ppt, dtype=jnp.int32)[None, :]
             + (ppt // 16) * jnp.arange(16, dtype=jnp.int32)[:, None]) % ppt,
        (2, 16, ppt))
    dst = jnp.concatenate(
        [edge_index[1].astype(jnp.int32).reshape(2, 16, E // 32),
         pad_dst], axis=2).reshape(2, 16, CPT, 1, K)
    x_p = jnp.pad(x, ((0, NP - N), (0, 0)))
    batch_p = jnp.pad(batch.astype(jnp.int32), (0, NP - N),
                      constant_values=G).reshape(NT, 1, TN)

    deg = _deg_call(dst)
    dinv = _dinv_call(deg).reshape(NP, 1)

    y, yh = _prep_call(x_p, W1, dinv)
    p = _agg_call(y, yh, src, dst)
    y, yh = _mid_call(p, dinv, b1.reshape(1, D), W2)
    p = _agg_call(y, yh, src, dst)
    y, yh = _mid_call(p, dinv, b2.reshape(1, D), W3)
    p = _agg_call(y, yh, src, dst)
    return _final_call(p, dinv, b3.reshape(1, D), batch_p, W_lin1,
                       b_lin1.reshape(1, D // 2), W_lin2,
                       b_lin2.reshape(1, 1))


# speedup vs baseline: 3.9111x; 1.0156x over previous
import functools

import jax
import jax.numpy as jnp
from jax import lax
from jax.experimental import pallas as pl
from jax.experimental.pallas import tpu as pltpu
from jax.experimental.pallas import tpu_sc as plsc

N = 10000
E = 320000
D = 128
G = 32

NP = 10240
TN = 1024
NT = NP // TN
K = 128
EP = 327680
EPT = EP // 32
CPT = EPT // K
NBUF = 2
RPT = NP // 16

_mesh = plsc.VectorSubcoreMesh(core_axis_name="c", subcore_axis_name="s")


def _deg_body(ei_hbm, out_hbm, est, onesv, zbuf, deg_sh):
    c = lax.axis_index("c")
    s = lax.axis_index("s")
    for j in range(RPT // 16):
        zbuf[pl.ds(j * 16, 16)] = jnp.zeros((16,), jnp.float32)
    for j in range(K // 16):
        onesv[pl.ds(j * 16, 16)] = jnp.ones((16,), jnp.float32)
    pltpu.sync_copy(zbuf, deg_sh.at[pl.ds(s * RPT, RPT)])
    pltpu.sync_copy(ei_hbm.at[c, s], est)
    plsc.subcore_barrier()

    def body(i, carry):
        pltpu.sync_copy(onesv, deg_sh.at[est.at[i, 0]], add=True)
        return carry

    lax.fori_loop(0, CPT, body, 0)
    plsc.subcore_barrier()
    pltpu.sync_copy(deg_sh.at[pl.ds(s * RPT, RPT)], out_hbm.at[c, pl.ds(s * RPT, RPT)])


_deg_call = pl.kernel(
    _deg_body,
    out_type=jax.ShapeDtypeStruct((2, NP), jnp.float32),
    mesh=_mesh,
    scratch_types=[
        pltpu.VMEM((CPT, 1, K), jnp.int32),
        pltpu.VMEM((K,), jnp.float32),
        pltpu.VMEM((RPT,), jnp.float32),
        pltpu.VMEM_SHARED((NP,), jnp.float32),
    ],
)


def _agg_body(y_hbm, src_hbm, dst_hbm, p_hbm, src_st, dstv, rows,
              acc_sh, gsem, dsem):
    c = lax.axis_index("c")
    s = lax.axis_index("s")
    r0 = s * RPT
    pltpu.sync_copy(src_hbm.at[c, s], src_st)

    @pl.when(c == 0)
    def _():
        pltpu.sync_copy(y_hbm.at[pl.ds(r0, RPT)], acc_sh.at[pl.ds(r0, RPT)])

    @pl.when(c == 1)
    def _():
        def zr(j, t):
            for cc in range(8):
                rows[0, j, pl.ds(cc * 16, 16)] = jnp.zeros((16,), jnp.float32)
            return t

        lax.fori_loop(0, K, zr, 0)

        def zc(j, t):
            pltpu.sync_copy(rows.at[0], acc_sh.at[pl.ds(r0 + j * K, K)])
            return t

        lax.fori_loop(0, RPT // K, zc, 0)

    plsc.subcore_barrier()

    for b in range(NBUF):
        pltpu.async_copy(dst_hbm.at[c, s, b], dstv.at[b], dsem.at[b])
        pltpu.async_copy(y_hbm.at[src_st.at[b, 0]], rows.at[b], gsem.at[b])

    def outer(g, carry):
        i0 = g * NBUF
        for b in range(NBUF):
            i = i0 + b
            pltpu.make_async_copy(y_hbm.at[src_st.at[i, 0]], rows.at[b],
                                  gsem.at[b]).wait()
            pltpu.make_async_copy(dst_hbm.at[c, s, i], dstv.at[b],
                                  dsem.at[b]).wait()
            pltpu.sync_copy(rows.at[b], acc_sh.at[dstv.at[b, 0]], add=True)

            @pl.when(i + NBUF < CPT)
            def _():
                pltpu.async_copy(dst_hbm.at[c, s, i + NBUF], dstv.at[b],
                                 dsem.at[b])
                pltpu.async_copy(y_hbm.at[src_st.at[i + NBUF, 0]], rows.at[b],
                                 gsem.at[b])

        return carry

    lax.fori_loop(0, CPT // NBUF, outer, 0)
    plsc.subcore_barrier()
    pltpu.sync_copy(acc_sh.at[pl.ds(r0, RPT)], p_hbm.at[c, pl.ds(r0, RPT)])


_agg_call = pl.kernel(
    _agg_body,
    out_type=jax.ShapeDtypeStruct((2, NP, D), jnp.float32),
    mesh=_mesh,
    scratch_types=[
        pltpu.VMEM((CPT, 1, K), jnp.int32),
        pltpu.VMEM((NBUF, 1, K), jnp.int32),
        pltpu.VMEM((NBUF, K, D), jnp.float32),
        pltpu.VMEM_SHARED((NP, D), jnp.float32),
        pltpu.SemaphoreType.DMA((NBUF,)),
        pltpu.SemaphoreType.DMA((NBUF,)),
    ],
)


def _gelu(x):
    return 0.5 * x * (1.0 + lax.erf(x * 0.7071067811865476))


def _dinv_body(deg_ref, out_ref):
    out_ref[...] = lax.rsqrt(deg_ref[0:1, :] + deg_ref[1:2, :] + 1.0)


_dinv_call = pl.pallas_call(
    _dinv_body,
    out_shape=jax.ShapeDtypeStruct((1, NP), jnp.float32),
)


def _prep_body(x_ref, w_ref, dinv_ref, y_ref):
    y_ref[...] = dinv_ref[...] * jnp.dot(x_ref[...], w_ref[...],
                                         preferred_element_type=jnp.float32)


_prep_call = pl.pallas_call(
    _prep_body,
    grid=(NT,),
    in_specs=[
        pl.BlockSpec((TN, D), lambda i: (i, 0)),
        pl.BlockSpec((D, D), lambda i: (0, 0)),
        pl.BlockSpec((TN, 1), lambda i: (i, 0)),
    ],
    out_specs=pl.BlockSpec((TN, D), lambda i: (i, 0)),
    out_shape=jax.ShapeDtypeStruct((NP, D), jnp.float32),
)


def _mid_body(p_ref, dinv_ref, b_ref, w_ref, y_ref):
    dv = dinv_ref[...]
    h = _gelu(dv * (p_ref[0] + p_ref[1]) + b_ref[...])
    y_ref[...] = dv * jnp.dot(h, w_ref[...], preferred_element_type=jnp.float32)


_mid_call = pl.pallas_call(
    _mid_body,
    grid=(NT,),
    in_specs=[
        pl.BlockSpec((2, TN, D), lambda i: (0, i, 0)),
        pl.BlockSpec((TN, 1), lambda i: (i, 0)),
        pl.BlockSpec((1, D), lambda i: (0, 0)),
        pl.BlockSpec((D, D), lambda i: (0, 0)),
    ],
    out_specs=pl.BlockSpec((TN, D), lambda i: (i, 0)),
    out_shape=jax.ShapeDtypeStruct((NP, D), jnp.float32),
)


def _final_body(p_ref, dinv_ref, b3_ref, batch_ref, wl1_ref, bl1_ref,
                wl2_ref, bl2_ref, out_ref, acc_s, acc_c):
    i = pl.program_id(0)
    dv = dinv_ref[...]
    h = _gelu(dv * (p_ref[0] + p_ref[1]) + b3_ref[...])
    bt = batch_ref[0]
    gids = lax.broadcasted_iota(jnp.int32, (G, TN), 0)
    oh = (bt == gids).astype(jnp.float32)

    @pl.when(i == 0)
    def _():
        acc_s[...] = jnp.zeros_like(acc_s)
        acc_c[...] = jnp.zeros_like(acc_c)

    acc_s[...] += jnp.dot(oh, h, preferred_element_type=jnp.float32)
    acc_c[...] += jnp.sum(oh, axis=1, keepdims=True)

    @pl.when(i == pl.num_programs(0) - 1)
    def _():
        pooled = acc_s[...] / jnp.maximum(acc_c[...], 1.0)
        t = jnp.dot(pooled, wl1_ref[...],
                    preferred_element_type=jnp.float32) + bl1_ref[...]
        t = jnp.where(t > 0, t, jnp.exp(jnp.minimum(t, 0.0)) - 1.0)
        out_ref[...] = jnp.dot(t, wl2_ref[...],
                               preferred_element_type=jnp.float32) + bl2_ref[...]


_final_call = pl.pallas_call(
    _final_body,
    grid=(NT,),
    in_specs=[
        pl.BlockSpec((2, TN, D), lambda i: (0, i, 0)),
        pl.BlockSpec((TN, 1), lambda i: (i, 0)),
        pl.BlockSpec((1, D), lambda i: (0, 0)),
        pl.BlockSpec((1, 1, TN), lambda i: (i, 0, 0)),
        pl.BlockSpec((D, D // 2), lambda i: (0, 0)),
        pl.BlockSpec((1, D // 2), lambda i: (0, 0)),
        pl.BlockSpec((D // 2, 1), lambda i: (0, 0)),
        pl.BlockSpec((1, 1), lambda i: (0, 0)),
    ],
    out_specs=pl.BlockSpec((G, 1), lambda i: (0, 0)),
    out_shape=jax.ShapeDtypeStruct((G, 1), jnp.float32),
    scratch_shapes=[
        pltpu.VMEM((G, D), jnp.float32),
        pltpu.VMEM((G, 1), jnp.float32),
    ],
)


def kernel(x, edge_index, batch, edge_weight, W1, b1, W2, b2, W3, b3,
           W_lin1, b_lin1, W_lin2, b_lin2):
    ppt = EPT - E // 32
    pad_src = jnp.broadcast_to(
        (jnp.arange(ppt, dtype=jnp.int32)[None, :] * 37
         + 613 * jnp.arange(16, dtype=jnp.int32)[:, None]) % N,
        (2, 16, ppt))
    src = jnp.concatenate(
        [edge_index[0].astype(jnp.int32).reshape(2, 16, E // 32),
         pad_src], axis=2).reshape(2, 16, CPT, 1, K)
    pad_dst = jnp.broadcast_to(
        N + (jnp.arange(ppt, dtype=jnp.int32)[None, :]
             + (ppt // 16) * jnp.arange(16, dtype=jnp.int32)[:, None]) % ppt,
        (2, 16, ppt))
    dst = jnp.concatenate(
        [edge_index[1].astype(jnp.int32).reshape(2, 16, E // 32),
         pad_dst], axis=2).reshape(2, 16, CPT, 1, K)
    x_p = jnp.pad(x, ((0, NP - N), (0, 0)))
    batch_p = jnp.pad(batch.astype(jnp.int32), (0, NP - N),
                      constant_values=G).reshape(NT, 1, TN)

    deg = _deg_call(dst)
    dinv = _dinv_call(deg).reshape(NP, 1)

    y = _prep_call(x_p, W1, dinv)
    p = _agg_call(y, src, dst)
    y = _mid_call(p, dinv, b1.reshape(1, D), W2)
    p = _agg_call(y, src, dst)
    y = _mid_call(p, dinv, b2.reshape(1, D), W3)
    p = _agg_call(y, src, dst)
    return _final_call(p, dinv, b3.reshape(1, D), batch_p, W_lin1,
                       b_lin1.reshape(1, D // 2), W_lin2,
                       b_lin2.reshape(1, 1))
